# baseline (speedup 1.0000x reference)
import functools

import jax
import jax.numpy as jnp
from jax import lax
from jax.experimental import pallas as pl
from jax.experimental.pallas import tpu as pltpu
from jax.experimental.pallas import tpu_sc as plsc

N = 10000
E = 320000
D = 128
NP = 10240
NC, NS = 2, 16
NW = NC * NS
EPT = E // NW
CH = 80
NCH = EPT // CH
SCH = 25
NSUP = NCH // SCH
DW = 16
RPT = NP // NS

_mesh = plsc.VectorSubcoreMesh(core_axis_name="c", subcore_axis_name="s")


def _wid():
    return lax.axis_index("s") * NC + lax.axis_index("c")


@functools.partial(
    pl.kernel,
    out_type=jax.ShapeDtypeStruct((NW, NP), jnp.float32),
    mesh=_mesh,
    scratch_types=[
        pltpu.VMEM((EPT,), jnp.int32),
        pltpu.VMEM((NP,), jnp.float32),
    ],
    compiler_params=pltpu.CompilerParams(needs_layout_passes=False),
)
def _deg(dst2_hbm, zeros_hbm, out_hbm, didx_v, hist_v):
    wid = _wid()
    pltpu.sync_copy(dst2_hbm.at[wid], didx_v)
    pltpu.sync_copy(zeros_hbm, hist_v)
    ones = jnp.full((16,), 1.0, jnp.float32)

    def body(j, carry):
        base = pl.multiple_of(j * 16, 16)
        plsc.addupdate_scatter(hist_v, [didx_v[pl.ds(base, 16)]], ones)
        return carry

    lax.fori_loop(0, EPT // 16, body, 0)
    pltpu.sync_copy(hist_v, out_hbm.at[wid])


@functools.partial(
    pl.kernel,
    out_type=jax.ShapeDtypeStruct((NC, NP, D), jnp.float32),
    mesh=_mesh,
    scratch_types=[
        pltpu.VMEM((SCH, CH), jnp.int32),
        pltpu.VMEM((SCH, CH), jnp.int32),
        pltpu.VMEM((CH, D), jnp.float32),
        pltpu.VMEM((CH, D), jnp.float32),
        pltpu.VMEM_SHARED((NP, D), jnp.float32),
        pltpu.SemaphoreType.DMA,
        pltpu.SemaphoreType.DMA,
    ],
)
def _agg(g_hbm, src4_hbm, dst4_hbm, zeros_hbm, out_hbm, sidx_v, didx_v,
         rows_a, rows_b, acc_sh, sem_a, sem_b):
    cid = lax.axis_index("c")
    sid = lax.axis_index("s")
    wid = _wid()
    pltpu.sync_copy(zeros_hbm.at[pl.ds(sid * RPT, RPT)],
                    acc_sh.at[pl.ds(sid * RPT, RPT)])
    plsc.subcore_barrier()

    def superstep(s, carry):
        pltpu.sync_copy(src4_hbm.at[wid, s], sidx_v)
        pltpu.sync_copy(dst4_hbm.at[wid, s], didx_v)
        pltpu.async_copy(g_hbm.at[sidx_v.at[0]], rows_a, sem_a)

        def body(p, carry2):
            j = p * 2
            pltpu.make_async_copy(g_hbm.at[sidx_v.at[j]], rows_a, sem_a).wait()
            pltpu.async_copy(g_hbm.at[sidx_v.at[j + 1]], rows_b, sem_b)
            pltpu.sync_copy(rows_a, acc_sh.at[didx_v.at[j]], add=True)
            pltpu.make_async_copy(g_hbm.at[sidx_v.at[j + 1]], rows_b,
                                  sem_b).wait()
            pltpu.async_copy(g_hbm.at[sidx_v.at[j + 2]], rows_a, sem_a)
            pltpu.sync_copy(rows_b, acc_sh.at[didx_v.at[j + 1]], add=True)
            return carry2

        lax.fori_loop(0, (SCH - 1) // 2, body, 0)
        pltpu.make_async_copy(g_hbm.at[sidx_v.at[SCH - 1]], rows_a,
                              sem_a).wait()
        pltpu.sync_copy(rows_a, acc_sh.at[didx_v.at[SCH - 1]], add=True)
        return carry

    lax.fori_loop(0, NSUP, superstep, 0)
    plsc.subcore_barrier()
    pltpu.sync_copy(acc_sh.at[pl.ds(sid * RPT, RPT)],
                    out_hbm.at[cid, pl.ds(sid * RPT, RPT)])


@functools.partial(
    pl.kernel,
    out_type=jax.ShapeDtypeStruct((NC, 2, NP), jnp.float32),
    mesh=_mesh,
    scratch_types=[
        pltpu.VMEM((SCH, CH), jnp.int32),
        pltpu.VMEM((SCH, CH), jnp.int32),
        pltpu.VMEM((CH, D), jnp.float32),
        pltpu.VMEM((CH, D), jnp.float32),
        pltpu.VMEM((CH, D), jnp.float32),
        pltpu.VMEM((RPT,), jnp.float32),
        pltpu.VMEM((RPT,), jnp.float32),
        pltpu.VMEM((2, D), jnp.float32),
        pltpu.VMEM_SHARED((NP, D), jnp.float32),
        pltpu.SemaphoreType.DMA,
        pltpu.SemaphoreType.DMA,
    ],
    compiler_params=pltpu.CompilerParams(needs_layout_passes=False),
)
def _aggf(g_hbm, src4_hbm, dst4_hbm, zeros_hbm, fw_hbm, out_hbm, sidx_v,
          didx_v, rows_a, rows_b, stage_v, u_loc, v_loc, w_s, acc_sh,
          sem_a, sem_b):
    cid = lax.axis_index("c")
    sid = lax.axis_index("s")
    wid = _wid()
    pltpu.sync_copy(zeros_hbm.at[pl.ds(sid * RPT, RPT)],
                    acc_sh.at[pl.ds(sid * RPT, RPT)])
    pltpu.sync_copy(fw_hbm, w_s)
    plsc.subcore_barrier()

    def superstep(s, carry):
        pltpu.sync_copy(src4_hbm.at[wid, s], sidx_v)
        pltpu.sync_copy(dst4_hbm.at[wid, s], didx_v)
        pltpu.async_copy(g_hbm.at[sidx_v.at[0]], rows_a, sem_a)

        def body(p, carry2):
            j = p * 2
            pltpu.make_async_copy(g_hbm.at[sidx_v.at[j]], rows_a, sem_a).wait()
            pltpu.async_copy(g_hbm.at[sidx_v.at[j + 1]], rows_b, sem_b)
            pltpu.sync_copy(rows_a, acc_sh.at[didx_v.at[j]], add=True)
            pltpu.make_async_copy(g_hbm.at[sidx_v.at[j + 1]], rows_b,
                                  sem_b).wait()
            pltpu.async_copy(g_hbm.at[sidx_v.at[j + 2]], rows_a, sem_a)
            pltpu.sync_copy(rows_b, acc_sh.at[didx_v.at[j + 1]], add=True)
            return carry2

        lax.fori_loop(0, (SCH - 1) // 2, body, 0)
        pltpu.make_async_copy(g_hbm.at[sidx_v.at[SCH - 1]], rows_a,
                              sem_a).wait()
        pltpu.sync_copy(rows_a, acc_sh.at[didx_v.at[SCH - 1]], add=True)
        return carry

    lax.fori_loop(0, NSUP, superstep, 0)
    plsc.subcore_barrier()

    lanes = jnp.arange(16, dtype=jnp.int32)
    zero16 = jnp.zeros((16,), jnp.float32)

    def piece(b, carry):
        pltpu.sync_copy(acc_sh.at[pl.ds(sid * RPT + b * CH, CH)], stage_v)
        for r0 in range(0, CH, 16):
            rows = lanes + r0

            def kbody(kk, uv):
                u16, v16 = uv
                kbase = kk * 16
                wu = w_s[0, pl.ds(pl.multiple_of(kbase, 16), 16)]
                wv = w_s[1, pl.ds(pl.multiple_of(kbase, 16), 16)]
                for k8 in range(16):
                    col = plsc.load_gather(stage_v, [rows, lanes * 0 + kbase + k8])
                    sel = jnp.full((16,), k8, jnp.int32)
                    u16 = u16 + col * wu[sel]
                    v16 = v16 + col * wv[sel]
                return (u16, v16)

            u16, v16 = lax.fori_loop(0, D // 16, kbody, (zero16, zero16))
            pos = pl.multiple_of(b * CH + r0, 16)
            u_loc[pl.ds(pos, 16)] = u16
            v_loc[pl.ds(pos, 16)] = v16
        return carry

    lax.fori_loop(0, RPT // CH, piece, 0)
    pltpu.sync_copy(u_loc, out_hbm.at[cid, 0, pl.ds(sid * RPT, RPT)])
    pltpu.sync_copy(v_loc, out_hbm.at[cid, 1, pl.ds(sid * RPT, RPT)])


@functools.partial(
    pl.kernel,
    out_type=jax.ShapeDtypeStruct((E,), jnp.float32),
    mesh=_mesh,
    scratch_types=[
        pltpu.VMEM((NP,), jnp.float32),
        pltpu.VMEM((NP,), jnp.float32),
        pltpu.VMEM((NP,), jnp.float32),
        pltpu.VMEM((NP,), jnp.float32),
        pltpu.VMEM((NP,), jnp.float32),
        pltpu.VMEM((NP,), jnp.float32),
        pltpu.VMEM((NP,), jnp.float32),
        pltpu.VMEM((EPT,), jnp.int32),
        pltpu.VMEM((EPT,), jnp.int32),
        pltpu.VMEM((EPT,), jnp.float32),
    ],
    compiler_params=pltpu.CompilerParams(needs_layout_passes=False),
)
def _rate(dinv_hbm, uva_hbm, ug_hbm, vg_hbm, src2_hbm, dst2_hbm, out_hbm,
          ua_v, ub_v, va_v, vb_v, ug_v, vg_v, dv_v, s_v, d_v, o_v):
    wid = _wid()
    pltpu.sync_copy(uva_hbm.at[0, 0], ua_v)
    pltpu.sync_copy(uva_hbm.at[1, 0], ub_v)
    pltpu.sync_copy(uva_hbm.at[0, 1], va_v)
    pltpu.sync_copy(uva_hbm.at[1, 1], vb_v)
    pltpu.sync_copy(ug_hbm, ug_v)
    pltpu.sync_copy(vg_hbm, vg_v)
    pltpu.sync_copy(dinv_hbm, dv_v)
    pltpu.sync_copy(src2_hbm.at[wid], s_v)
    pltpu.sync_copy(dst2_hbm.at[wid], d_v)

    def combine(i, carry):
        sl = pl.ds(pl.multiple_of(i * 16, 16), 16)
        dv = dv_v[sl]
        ua_v[sl] = dv * (ua_v[sl] + ub_v[sl]) + ug_v[sl]
        va_v[sl] = dv * (va_v[sl] + vb_v[sl]) + vg_v[sl]
        return carry

    lax.fori_loop(0, NP // 16, combine, 0)

    def body(j, carry):
        base = pl.multiple_of(j * 16, 16)
        si = s_v[pl.ds(base, 16)]
        di = d_v[pl.ds(base, 16)]
        a = plsc.load_gather(ua_v, [si])
        b = plsc.load_gather(va_v, [di])
        z = a + b
        o_v[pl.ds(base, 16)] = 4.0 / (1.0 + jnp.exp(-z)) + 1.0
        return carry

    lax.fori_loop(0, EPT // 16, body, 0)
    pltpu.sync_copy(o_v, out_hbm.at[pl.ds(wid * EPT, EPT)])


_R = 1024
_G = NP // _R


def _mm1_body(h_ref, x_ref, w_ref, g_ref, dv_ref):
    deg = 1.0 + jnp.sum(h_ref[...], axis=0)
    dinv = lax.rsqrt(deg)
    h = jnp.dot(x_ref[...], w_ref[...], preferred_element_type=jnp.float32)
    g_ref[...] = h * dinv
    dv_ref[...] = dinv


_mm1 = pl.pallas_call(
    _mm1_body,
    grid=(_G,),
    in_specs=[
        pl.BlockSpec((NW, _R, 1), lambda i: (0, i, 0)),
        pl.BlockSpec((_R, D), lambda i: (i, 0)),
        pl.BlockSpec((D, D), lambda i: (0, 0)),
    ],
    out_specs=[
        pl.BlockSpec((_R, D), lambda i: (i, 0)),
        pl.BlockSpec((_R, 1), lambda i: (i, 0)),
    ],
    out_shape=[
        jax.ShapeDtypeStruct((NP, D), jnp.float32),
        jax.ShapeDtypeStruct((NP, 1), jnp.float32),
    ],
)


def _mid_body(dv_ref, acc_ref, g1_ref, b1_ref, w2_ref, b2_ref, fw_ref,
              fb_ref, g2_ref, ug_ref, vg_ref):
    dinv = dv_ref[...]
    z = (acc_ref[0] + acc_ref[1] + g1_ref[...]) * dinv + b1_ref[...]
    t = jnp.where(z > 0, z, jnp.exp(jnp.minimum(z, 0.0)) - 1.0)
    tw = jnp.dot(t, w2_ref[...], preferred_element_type=jnp.float32)
    g2_ref[...] = tw * dinv
    d2 = dinv * dinv
    w0 = fw_ref[0:1, :]
    w1 = fw_ref[1:2, :]
    b2 = b2_ref[...]
    ug_ref[...] = (jnp.sum(tw * w0, axis=1, keepdims=True) * d2
                   + jnp.sum(b2 * w0) + fb_ref[0, 0])
    vg_ref[...] = (jnp.sum(tw * w1, axis=1, keepdims=True) * d2
                   + jnp.sum(b2 * w1))


_mid = pl.pallas_call(
    _mid_body,
    grid=(_G,),
    in_specs=[
        pl.BlockSpec((_R, 1), lambda i: (i, 0)),
        pl.BlockSpec((NC, _R, D), lambda i: (0, i, 0)),
        pl.BlockSpec((_R, D), lambda i: (i, 0)),
        pl.BlockSpec((1, D), lambda i: (0, 0)),
        pl.BlockSpec((D, D), lambda i: (0, 0)),
        pl.BlockSpec((1, D), lambda i: (0, 0)),
        pl.BlockSpec((2, D), lambda i: (0, 0)),
        pl.BlockSpec((1, 1), lambda i: (0, 0)),
    ],
    out_specs=[
        pl.BlockSpec((_R, D), lambda i: (i, 0)),
        pl.BlockSpec((_R, 1), lambda i: (i, 0)),
        pl.BlockSpec((_R, 1), lambda i: (i, 0)),
    ],
    out_shape=[
        jax.ShapeDtypeStruct((NP, D), jnp.float32),
        jax.ShapeDtypeStruct((NP, 1), jnp.float32),
        jax.ShapeDtypeStruct((NP, 1), jnp.float32),
    ],
)


def kernel(x, edge_index, W1, b1, W2, b2, fc_w, fc_b):
    src = edge_index[0]
    dst = edge_index[1]
    src4 = src.reshape(NW, NSUP, SCH, CH)
    dst4 = dst.reshape(NW, NSUP, SCH, CH)
    src2 = src.reshape(NW, EPT)
    dst2 = dst.reshape(NW, EPT)
    x_pad = jnp.pad(x, ((0, NP - N), (0, 0)))
    zeros2 = jnp.zeros((NP, D), jnp.float32)
    zeros1 = jnp.zeros((NP,), jnp.float32)

    hist = _deg(dst2, zeros1)
    g1, dinv = _mm1(hist[:, :, None], x_pad, W1)
    acc1 = _agg(g1, src4, dst4, zeros2)
    fcw2 = jnp.concatenate([fc_w[:D].reshape(1, D), fc_w[D:].reshape(1, D)],
                           axis=0)
    g2, ug, vg = _mid(dinv, acc1, g1, b1.reshape(1, D), W2,
                      b2.reshape(1, D), fcw2, fc_b.reshape(1, 1))
    uva = _aggf(g2, src4, dst4, zeros2, fcw2)
    return _rate(dinv.reshape(NP), uva, ug.reshape(NP), vg.reshape(NP),
                 src2, dst2)


# speedup vs baseline: 1.0550x; 1.0550x over previous
import functools

import jax
import jax.numpy as jnp
from jax import lax
from jax.experimental import pallas as pl
from jax.experimental.pallas import tpu as pltpu
from jax.experimental.pallas import tpu_sc as plsc

N = 10000
E = 320000
D = 128
NP = 10240
NC, NS = 2, 16
NW = NC * NS
EPT = E // NW
CH = 80
NCH = EPT // CH
SCH = 25
NSUP = NCH // SCH
DW = 16
RPT = NP // NS

_mesh = plsc.VectorSubcoreMesh(core_axis_name="c", subcore_axis_name="s")


def _wid():
    return lax.axis_index("s") * NC + lax.axis_index("c")


@functools.partial(
    pl.kernel,
    out_type=jax.ShapeDtypeStruct((NW, NP), jnp.float32),
    mesh=_mesh,
    scratch_types=[
        pltpu.VMEM((EPT,), jnp.int32),
        pltpu.VMEM((NP,), jnp.float32),
    ],
    compiler_params=pltpu.CompilerParams(needs_layout_passes=False),
)
def _deg(dst2_hbm, zeros_hbm, out_hbm, didx_v, hist_v):
    wid = _wid()
    pltpu.sync_copy(dst2_hbm.at[wid], didx_v)
    pltpu.sync_copy(zeros_hbm, hist_v)
    ones = jnp.full((16,), 1.0, jnp.float32)

    def body(j, carry):
        base = pl.multiple_of(j * 16, 16)
        plsc.addupdate_scatter(hist_v, [didx_v[pl.ds(base, 16)]], ones)
        return carry

    lax.fori_loop(0, EPT // 16, body, 0)
    pltpu.sync_copy(hist_v, out_hbm.at[wid])


@functools.partial(
    pl.kernel,
    out_type=jax.ShapeDtypeStruct((NC, NP, D), jnp.float32),
    mesh=_mesh,
    scratch_types=[
        pltpu.VMEM((SCH, CH), jnp.int32),
        pltpu.VMEM((SCH, CH), jnp.int32),
        pltpu.VMEM((CH, D), jnp.float32),
        pltpu.VMEM((CH, D), jnp.float32),
        pltpu.VMEM_SHARED((NP, D), jnp.float32),
        pltpu.SemaphoreType.DMA,
        pltpu.SemaphoreType.DMA,
    ],
)
def _agg(g_hbm, src4_hbm, dst4_hbm, zeros_hbm, out_hbm, sidx_v, didx_v,
         rows_a, rows_b, acc_sh, sem_a, sem_b):
    cid = lax.axis_index("c")
    sid = lax.axis_index("s")
    wid = _wid()
    pltpu.sync_copy(zeros_hbm.at[pl.ds(sid * RPT, RPT)],
                    acc_sh.at[pl.ds(sid * RPT, RPT)])
    plsc.subcore_barrier()

    def superstep(s, carry):
        pltpu.sync_copy(src4_hbm.at[wid, s], sidx_v)
        pltpu.sync_copy(dst4_hbm.at[wid, s], didx_v)
        pltpu.async_copy(g_hbm.at[sidx_v.at[0]], rows_a, sem_a)

        def body(p, carry2):
            j = p * 2
            pltpu.make_async_copy(g_hbm.at[sidx_v.at[j]], rows_a, sem_a).wait()
            pltpu.async_copy(g_hbm.at[sidx_v.at[j + 1]], rows_b, sem_b)
            pltpu.sync_copy(rows_a, acc_sh.at[didx_v.at[j]], add=True)
            pltpu.make_async_copy(g_hbm.at[sidx_v.at[j + 1]], rows_b,
                                  sem_b).wait()
            pltpu.async_copy(g_hbm.at[sidx_v.at[j + 2]], rows_a, sem_a)
            pltpu.sync_copy(rows_b, acc_sh.at[didx_v.at[j + 1]], add=True)
            return carry2

        lax.fori_loop(0, (SCH - 1) // 2, body, 0)
        pltpu.make_async_copy(g_hbm.at[sidx_v.at[SCH - 1]], rows_a,
                              sem_a).wait()
        pltpu.sync_copy(rows_a, acc_sh.at[didx_v.at[SCH - 1]], add=True)
        return carry

    lax.fori_loop(0, NSUP, superstep, 0)
    plsc.subcore_barrier()
    pltpu.sync_copy(acc_sh.at[pl.ds(sid * RPT, RPT)],
                    out_hbm.at[cid, pl.ds(sid * RPT, RPT)])


@functools.partial(
    pl.kernel,
    out_type=jax.ShapeDtypeStruct((E,), jnp.float32),
    mesh=_mesh,
    scratch_types=[
        pltpu.VMEM((NP // D, D), jnp.float32),
        pltpu.VMEM((NP // D, D), jnp.float32),
        pltpu.VMEM((EPT,), jnp.int32),
        pltpu.VMEM((EPT,), jnp.int32),
        pltpu.VMEM((EPT,), jnp.float32),
    ],
    compiler_params=pltpu.CompilerParams(needs_layout_passes=False),
)
def _rate(u_hbm, v_hbm, src2_hbm, dst2_hbm, out_hbm, u_v, v_v, s_v, d_v,
          o_v):
    wid = _wid()
    pltpu.sync_copy(u_hbm, u_v)
    pltpu.sync_copy(v_hbm, v_v)
    pltpu.sync_copy(src2_hbm.at[wid], s_v)
    pltpu.sync_copy(dst2_hbm.at[wid], d_v)

    def body(j, carry):
        base = pl.multiple_of(j * 16, 16)
        si = s_v[pl.ds(base, 16)]
        di = d_v[pl.ds(base, 16)]
        a = plsc.load_gather(u_v, [si >> 7, si & 127])
        b = plsc.load_gather(v_v, [di >> 7, di & 127])
        z = a + b
        o_v[pl.ds(base, 16)] = 4.0 / (1.0 + jnp.exp(-z)) + 1.0
        return carry

    lax.fori_loop(0, EPT // 16, body, 0)
    pltpu.sync_copy(o_v, out_hbm.at[pl.ds(wid * EPT, EPT)])


_R = 1024
_G = NP // _R


def _mm1_body(h_ref, x_ref, w_ref, g_ref, dv_ref):
    deg = 1.0 + jnp.sum(h_ref[...], axis=0)
    dinv = lax.rsqrt(deg)
    h = jnp.dot(x_ref[...], w_ref[...], preferred_element_type=jnp.float32)
    g_ref[...] = h * dinv
    dv_ref[...] = dinv


_mm1 = pl.pallas_call(
    _mm1_body,
    grid=(_G,),
    in_specs=[
        pl.BlockSpec((NW, _R, 1), lambda i: (0, i, 0)),
        pl.BlockSpec((_R, D), lambda i: (i, 0)),
        pl.BlockSpec((D, D), lambda i: (0, 0)),
    ],
    out_specs=[
        pl.BlockSpec((_R, D), lambda i: (i, 0)),
        pl.BlockSpec((_R, 1), lambda i: (i, 0)),
    ],
    out_shape=[
        jax.ShapeDtypeStruct((NP, D), jnp.float32),
        jax.ShapeDtypeStruct((NP, 1), jnp.float32),
    ],
)


def _mid_body(dv_ref, acc_ref, g1_ref, b1_ref, w2_ref, g2_ref):
    dinv = dv_ref[...]
    z = (acc_ref[0] + acc_ref[1] + g1_ref[...]) * dinv + b1_ref[...]
    t = jnp.where(z > 0, z, jnp.exp(jnp.minimum(z, 0.0)) - 1.0)
    g2_ref[...] = jnp.dot(t, w2_ref[...],
                          preferred_element_type=jnp.float32) * dinv


_mid = pl.pallas_call(
    _mid_body,
    grid=(_G,),
    in_specs=[
        pl.BlockSpec((_R, 1), lambda i: (i, 0)),
        pl.BlockSpec((NC, _R, D), lambda i: (0, i, 0)),
        pl.BlockSpec((_R, D), lambda i: (i, 0)),
        pl.BlockSpec((1, D), lambda i: (0, 0)),
        pl.BlockSpec((D, D), lambda i: (0, 0)),
    ],
    out_specs=pl.BlockSpec((_R, D), lambda i: (i, 0)),
    out_shape=jax.ShapeDtypeStruct((NP, D), jnp.float32),
)


def _fin_body(dv_ref, acc_ref, g2_ref, b2_ref, w0_ref, w1_ref, fb_ref,
              u_ref, v_ref):
    dinv = dv_ref[...]
    h2 = (acc_ref[0] + acc_ref[1] + g2_ref[...]) * dinv + b2_ref[...]
    u_ref[...] = jnp.sum(h2 * w0_ref[...], axis=1, keepdims=True) + fb_ref[0, 0]
    v_ref[...] = jnp.sum(h2 * w1_ref[...], axis=1, keepdims=True)


_fin = pl.pallas_call(
    _fin_body,
    grid=(_G,),
    in_specs=[
        pl.BlockSpec((_R, 1), lambda i: (i, 0)),
        pl.BlockSpec((NC, _R, D), lambda i: (0, i, 0)),
        pl.BlockSpec((_R, D), lambda i: (i, 0)),
        pl.BlockSpec((1, D), lambda i: (0, 0)),
        pl.BlockSpec((1, D), lambda i: (0, 0)),
        pl.BlockSpec((1, D), lambda i: (0, 0)),
        pl.BlockSpec((1, 1), lambda i: (0, 0)),
    ],
    out_specs=[
        pl.BlockSpec((_R, 1), lambda i: (i, 0)),
        pl.BlockSpec((_R, 1), lambda i: (i, 0)),
    ],
    out_shape=[
        jax.ShapeDtypeStruct((NP, 1), jnp.float32),
        jax.ShapeDtypeStruct((NP, 1), jnp.float32),
    ],
)


def kernel(x, edge_index, W1, b1, W2, b2, fc_w, fc_b):
    src = edge_index[0]
    dst = edge_index[1]
    src4 = src.reshape(NW, NSUP, SCH, CH)
    dst4 = dst.reshape(NW, NSUP, SCH, CH)
    src2 = src.reshape(NW, EPT)
    dst2 = dst.reshape(NW, EPT)
    x_pad = jnp.pad(x, ((0, NP - N), (0, 0)))
    zeros2 = jnp.zeros((NP, D), jnp.float32)
    zeros1 = jnp.zeros((NP,), jnp.float32)

    hist = _deg(dst2, zeros1)
    g1, dinv = _mm1(hist[:, :, None], x_pad, W1)
    acc1 = _agg(g1, src4, dst4, zeros2)
    g2 = _mid(dinv, acc1, g1, b1.reshape(1, D), W2)
    acc2 = _agg(g2, src4, dst4, zeros2)
    u, v = _fin(dinv, acc2, g2, b2.reshape(1, D),
                fc_w[:D].reshape(1, D), fc_w[D:].reshape(1, D),
                fc_b.reshape(1, 1))
    return _rate(u.reshape(NP // D, D), v.reshape(NP // D, D), src2, dst2)


# speedup vs baseline: 1.0724x; 1.0165x over previous
import functools

import jax
import jax.numpy as jnp
from jax import lax
from jax.experimental import pallas as pl
from jax.experimental.pallas import tpu as pltpu
from jax.experimental.pallas import tpu_sc as plsc

N = 10000
E = 320000
D = 128
NP = 10240
NC, NS = 2, 16
NW = NC * NS
EPT = E // NW
CH = 80
NCH = EPT // CH
SCH = 25
NSUP = NCH // SCH
DW = 16
RPT = NP // NS

_mesh = plsc.VectorSubcoreMesh(core_axis_name="c", subcore_axis_name="s")


def _wid():
    return lax.axis_index("s") * NC + lax.axis_index("c")


@functools.partial(
    pl.kernel,
    out_type=jax.ShapeDtypeStruct((NW, NP), jnp.float32),
    mesh=_mesh,
    scratch_types=[
        pltpu.VMEM((EPT,), jnp.int32),
        pltpu.VMEM((NP,), jnp.float32),
    ],
    compiler_params=pltpu.CompilerParams(needs_layout_passes=False),
)
def _deg(dst2_hbm, zeros_hbm, out_hbm, didx_v, hist_v):
    wid = _wid()
    pltpu.sync_copy(dst2_hbm.at[wid], didx_v)
    pltpu.sync_copy(zeros_hbm, hist_v)
    ones = jnp.full((16,), 1.0, jnp.float32)

    def body(j, carry):
        base = pl.multiple_of(j * 16, 16)
        plsc.addupdate_scatter(hist_v, [didx_v[pl.ds(base, 16)]], ones)
        return carry

    lax.fori_loop(0, EPT // 16, body, 0)
    pltpu.sync_copy(hist_v, out_hbm.at[wid])


@functools.partial(
    pl.kernel,
    out_type=jax.ShapeDtypeStruct((NC, NP, D), jnp.float32),
    mesh=_mesh,
    scratch_types=[
        pltpu.VMEM((SCH, CH), jnp.int32),
        pltpu.VMEM((SCH, CH), jnp.int32),
        pltpu.VMEM((SCH, CH), jnp.int32),
        pltpu.VMEM((SCH, CH), jnp.int32),
        pltpu.VMEM((CH, D), jnp.float32),
        pltpu.VMEM((CH, D), jnp.float32),
        pltpu.VMEM_SHARED((NP, D), jnp.float32),
        pltpu.SemaphoreType.DMA,
        pltpu.SemaphoreType.DMA,
        pltpu.SemaphoreType.DMA,
    ],
)
def _agg(g_hbm, src4_hbm, dst4_hbm, zeros_hbm, out_hbm, sidx_a, didx_a,
         sidx_b, didx_b, rows_a, rows_b, acc_sh, sem_a, sem_b, sem_i):
    cid = lax.axis_index("c")
    sid = lax.axis_index("s")
    wid = _wid()
    pltpu.sync_copy(src4_hbm.at[wid, 0], sidx_a)
    pltpu.sync_copy(dst4_hbm.at[wid, 0], didx_a)
    pltpu.sync_copy(zeros_hbm.at[pl.ds(sid * RPT, RPT)],
                    acc_sh.at[pl.ds(sid * RPT, RPT)])
    plsc.subcore_barrier()

    def superstep(s, sidx_v, didx_v, sidx_n, didx_n):
        if s + 1 < NSUP:
            pf_s = pltpu.async_copy(src4_hbm.at[wid, s + 1], sidx_n, sem_i)
            pf_d = pltpu.async_copy(dst4_hbm.at[wid, s + 1], didx_n, sem_i)
        pltpu.async_copy(g_hbm.at[sidx_v.at[0]], rows_a, sem_a)

        def body(p, carry2):
            j = p * 2
            pltpu.make_async_copy(g_hbm.at[sidx_v.at[j]], rows_a, sem_a).wait()
            pltpu.async_copy(g_hbm.at[sidx_v.at[j + 1]], rows_b, sem_b)
            pltpu.sync_copy(rows_a, acc_sh.at[didx_v.at[j]], add=True)
            pltpu.make_async_copy(g_hbm.at[sidx_v.at[j + 1]], rows_b,
                                  sem_b).wait()
            pltpu.async_copy(g_hbm.at[sidx_v.at[j + 2]], rows_a, sem_a)
            pltpu.sync_copy(rows_b, acc_sh.at[didx_v.at[j + 1]], add=True)
            return carry2

        lax.fori_loop(0, (SCH - 1) // 2, body, 0)
        pltpu.make_async_copy(g_hbm.at[sidx_v.at[SCH - 1]], rows_a,
                              sem_a).wait()
        pltpu.sync_copy(rows_a, acc_sh.at[didx_v.at[SCH - 1]], add=True)
        if s + 1 < NSUP:
            pf_s.wait()
            pf_d.wait()

    for s in range(NSUP):
        if s % 2 == 0:
            superstep(s, sidx_a, didx_a, sidx_b, didx_b)
        else:
            superstep(s, sidx_b, didx_b, sidx_a, didx_a)
    plsc.subcore_barrier()
    pltpu.sync_copy(acc_sh.at[pl.ds(sid * RPT, RPT)],
                    out_hbm.at[cid, pl.ds(sid * RPT, RPT)])


@functools.partial(
    pl.kernel,
    out_type=jax.ShapeDtypeStruct((E,), jnp.float32),
    mesh=_mesh,
    scratch_types=[
        pltpu.VMEM((NP // D, D), jnp.float32),
        pltpu.VMEM((NP // D, D), jnp.float32),
        pltpu.VMEM((EPT,), jnp.int32),
        pltpu.VMEM((EPT,), jnp.int32),
        pltpu.VMEM((EPT,), jnp.float32),
    ],
    compiler_params=pltpu.CompilerParams(needs_layout_passes=False),
)
def _rate(u_hbm, v_hbm, src2_hbm, dst2_hbm, out_hbm, u_v, v_v, s_v, d_v,
          o_v):
    wid = _wid()
    pltpu.sync_copy(u_hbm, u_v)
    pltpu.sync_copy(v_hbm, v_v)
    pltpu.sync_copy(src2_hbm.at[wid], s_v)
    pltpu.sync_copy(dst2_hbm.at[wid], d_v)

    def body(j, carry):
        base = pl.multiple_of(j * 16, 16)
        si = s_v[pl.ds(base, 16)]
        di = d_v[pl.ds(base, 16)]
        a = plsc.load_gather(u_v, [si >> 7, si & 127])
        b = plsc.load_gather(v_v, [di >> 7, di & 127])
        z = a + b
        o_v[pl.ds(base, 16)] = 4.0 / (1.0 + jnp.exp(-z)) + 1.0
        return carry

    lax.fori_loop(0, EPT // 16, body, 0)
    pltpu.sync_copy(o_v, out_hbm.at[pl.ds(wid * EPT, EPT)])


_R = 1024
_G = NP // _R


def _mm1_body(h_ref, x_ref, w_ref, g_ref, dv_ref):
    deg = 1.0 + jnp.sum(h_ref[...], axis=0)
    dinv = lax.rsqrt(deg)
    h = jnp.dot(x_ref[...], w_ref[...], preferred_element_type=jnp.float32)
    g_ref[...] = h * dinv
    dv_ref[...] = dinv


_mm1 = pl.pallas_call(
    _mm1_body,
    grid=(_G,),
    in_specs=[
        pl.BlockSpec((NW, _R, 1), lambda i: (0, i, 0)),
        pl.BlockSpec((_R, D), lambda i: (i, 0)),
        pl.BlockSpec((D, D), lambda i: (0, 0)),
    ],
    out_specs=[
        pl.BlockSpec((_R, D), lambda i: (i, 0)),
        pl.BlockSpec((_R, 1), lambda i: (i, 0)),
    ],
    out_shape=[
        jax.ShapeDtypeStruct((NP, D), jnp.float32),
        jax.ShapeDtypeStruct((NP, 1), jnp.float32),
    ],
)


def _mid_body(dv_ref, acc_ref, g1_ref, b1_ref, w2_ref, g2_ref):
    dinv = dv_ref[...]
    z = (acc_ref[0] + acc_ref[1] + g1_ref[...]) * dinv + b1_ref[...]
    t = jnp.where(z > 0, z, jnp.exp(jnp.minimum(z, 0.0)) - 1.0)
    g2_ref[...] = jnp.dot(t, w2_ref[...],
                          preferred_element_type=jnp.float32) * dinv


_mid = pl.pallas_call(
    _mid_body,
    grid=(_G,),
    in_specs=[
        pl.BlockSpec((_R, 1), lambda i: (i, 0)),
        pl.BlockSpec((NC, _R, D), lambda i: (0, i, 0)),
        pl.BlockSpec((_R, D), lambda i: (i, 0)),
        pl.BlockSpec((1, D), lambda i: (0, 0)),
        pl.BlockSpec((D, D), lambda i: (0, 0)),
    ],
    out_specs=pl.BlockSpec((_R, D), lambda i: (i, 0)),
    out_shape=jax.ShapeDtypeStruct((NP, D), jnp.float32),
)


def _fin_body(dv_ref, acc_ref, g2_ref, b2_ref, w0_ref, w1_ref, fb_ref,
              u_ref, v_ref):
    dinv = dv_ref[...]
    h2 = (acc_ref[0] + acc_ref[1] + g2_ref[...]) * dinv + b2_ref[...]
    u_ref[...] = jnp.sum(h2 * w0_ref[...], axis=1, keepdims=True) + fb_ref[0, 0]
    v_ref[...] = jnp.sum(h2 * w1_ref[...], axis=1, keepdims=True)


_fin = pl.pallas_call(
    _fin_body,
    grid=(_G,),
    in_specs=[
        pl.BlockSpec((_R, 1), lambda i: (i, 0)),
        pl.BlockSpec((NC, _R, D), lambda i: (0, i, 0)),
        pl.BlockSpec((_R, D), lambda i: (i, 0)),
        pl.BlockSpec((1, D), lambda i: (0, 0)),
        pl.BlockSpec((1, D), lambda i: (0, 0)),
        pl.BlockSpec((1, D), lambda i: (0, 0)),
        pl.BlockSpec((1, 1), lambda i: (0, 0)),
    ],
    out_specs=[
        pl.BlockSpec((_R, 1), lambda i: (i, 0)),
        pl.BlockSpec((_R, 1), lambda i: (i, 0)),
    ],
    out_shape=[
        jax.ShapeDtypeStruct((NP, 1), jnp.float32),
        jax.ShapeDtypeStruct((NP, 1), jnp.float32),
    ],
)


def kernel(x, edge_index, W1, b1, W2, b2, fc_w, fc_b):
    src = edge_index[0]
    dst = edge_index[1]
    src4 = src.reshape(NW, NSUP, SCH, CH)
    dst4 = dst.reshape(NW, NSUP, SCH, CH)
    src2 = src.reshape(NW, EPT)
    dst2 = dst.reshape(NW, EPT)
    x_pad = jnp.pad(x, ((0, NP - N), (0, 0)))
    zeros2 = jnp.zeros((NP, D), jnp.float32)
    zeros1 = jnp.zeros((NP,), jnp.float32)

    hist = _deg(dst2, zeros1)
    g1, dinv = _mm1(hist[:, :, None], x_pad, W1)
    acc1 = _agg(g1, src4, dst4, zeros2)
    g2 = _mid(dinv, acc1, g1, b1.reshape(1, D), W2)
    acc2 = _agg(g2, src4, dst4, zeros2)
    u, v = _fin(dinv, acc2, g2, b2.reshape(1, D),
                fc_w[:D].reshape(1, D), fc_w[D:].reshape(1, D),
                fc_b.reshape(1, 1))
    return _rate(u.reshape(NP // D, D), v.reshape(NP // D, D), src2, dst2)
